# baseline (speedup 1.0000x reference)
import functools

import jax
import jax.numpy as jnp
from jax import lax
from jax.experimental import pallas as pl
from jax.experimental.pallas import tpu as pltpu
from jax.experimental.pallas import tpu_sc as plsc

MIN_NORM = 1e-15
EPS = 4e-3
MAX_NORM = 1e6


def _cosh(v):
    return 0.5 * (jnp.exp(v) + jnp.exp(-v))


def _sinh(v):
    return 0.5 * (jnp.exp(v) - jnp.exp(-v))


def _acosh(v):
    return jnp.log(v + jnp.sqrt(v * v - 1.0))


def _tailsq(v):
    return jnp.sum(v * v, axis=-1, keepdims=True) - v[:, 0:1] * v[:, 0:1]


def _proj_row(v, mask0):
    t = jnp.sqrt(jnp.clip(1.0 + _tailsq(v), EPS, None))
    return jnp.where(mask0, t, v)


def _expmap0_row(v, mask0):
    xn = jnp.sqrt(jnp.clip(_tailsq(v), MIN_NORM, None))
    w = (_sinh(xn) / xn) * v
    w = jnp.where(mask0, _cosh(xn), w)
    return _proj_row(w, mask0)


def _logmap0_row(v, mask0):
    yn = jnp.sqrt(jnp.clip(_tailsq(v), MIN_NORM, None))
    theta = jnp.clip(v[:, 0:1], 1.0 + EPS, None)
    w = (_acosh(theta) / yn) * v
    return jnp.where(mask0, 0.0, w)


def _encoder_body(x_ref, w_ref, b_ref, o_ref):
    x = x_ref[...]
    xn = jnp.sqrt(jnp.clip(jnp.sum(x * x, -1, keepdims=True), MIN_NORM, None))
    ys = (_sinh(xn) / xn) * x
    ysq = jnp.sum(ys * ys, -1, keepdims=True)
    t = jnp.sqrt(jnp.clip(1.0 + ysq, EPS, None))
    yn = jnp.sqrt(jnp.clip(ysq, MIN_NORM, None))
    theta = jnp.clip(t, 1.0 + EPS, None)
    ub = (_acosh(theta) / yn) * ys
    mm = lax.dot_general(ub, w_ref[...], (((1,), (1,)), ((), ())),
                         preferred_element_type=jnp.float32)
    mask0 = lax.broadcasted_iota(jnp.int32, mm.shape, 1) == 0
    res = _expmap0_row(mm, mask0)
    b = b_ref[...]
    bmask = lax.broadcasted_iota(jnp.int32, b.shape, 1) == 0
    hb = _expmap0_row(jnp.where(bmask, 0.0, b), bmask)
    lb = _logmap0_row(hb, bmask)
    x0 = res[:, 0:1]
    yn2 = jnp.sqrt(jnp.clip(_tailsq(res), MIN_NORM, None))
    alpha = jnp.sum(res * lb, -1, keepdims=True) / yn2
    vvec = jnp.where(mask0, -yn2, ((1.0 - x0) / yn2) * res)
    w = lb - alpha * vvec
    ux = jnp.sum(res * w, -1, keepdims=True) - x0 * w[:, 0:1]
    v0 = ux / jnp.clip(x0, EPS, None)
    u = jnp.where(mask0, v0, w)
    mdot = jnp.sum(u * u, -1, keepdims=True) - 2.0 * u[:, 0:1] * u[:, 0:1]
    normu = jnp.clip(jnp.sqrt(jnp.clip(mdot, EPS, None)), None, MAX_NORM)
    th = jnp.clip(normu, MIN_NORM, None)
    res2 = _proj_row(_cosh(th) * res + (_sinh(th) / th) * u, mask0)
    res2 = _proj_row(res2, mask0)
    xt = _logmap0_row(res2, mask0)
    o_ref[...] = xt


def _decoder_body(s_ref, wd_ref, bd_ref, o_ref):
    sup = s_ref[...]
    mask0 = lax.broadcasted_iota(jnp.int32, sup.shape, 1) == 0
    h = _expmap0_row(sup, mask0)
    lg = _logmap0_row(h, mask0)
    r = jnp.where(mask0, 0.0, jnp.maximum(lg, 0.0))
    h2 = _expmap0_row(r, mask0)
    hd = _logmap0_row(h2, mask0)
    out = lax.dot_general(hd, wd_ref[...], (((1,), (1,)), ((), ())),
                          preferred_element_type=jnp.float32) + bd_ref[...]
    m = jnp.max(out, -1, keepdims=True)
    z = out - m
    o_ref[...] = z - jnp.log(jnp.sum(jnp.exp(z), -1, keepdims=True))


def _seg_sum_sc(xt, src, dst, npad, chunks_per_tile, chunk, sup):
    half = npad // 2
    acc_rows = half + 128
    rows_per_tile = half // 16
    z_per_tile = acc_rows // 16
    n_sup = chunks_per_tile // sup
    mesh = plsc.VectorSubcoreMesh(core_axis_name="c", subcore_axis_name="s")

    @functools.partial(
        pl.kernel,
        out_type=jax.ShapeDtypeStruct((npad, 2, 128), jnp.float32),
        mesh=mesh,
        scratch_types=[
            pltpu.VMEM((sup, chunk), jnp.int32),
            pltpu.VMEM((sup, chunk), jnp.int32),
            pltpu.VMEM((chunk, 2, 128), jnp.float32),
            pltpu.VMEM_SHARED((acc_rows, 2, 128), jnp.float32),
            pltpu.SemaphoreType.DMA,
        ],
    )
    def seg_kernel(xt_hbm, src_hbm, dst_hbm, out_hbm, sidx, didx, buf0,
                   acc, sem0):
        c = lax.axis_index("c")
        s = lax.axis_index("s")
        zer = jnp.zeros((16,), jnp.float32)

        @pl.loop(0, chunk)
        def _(i):
            for j in range(16):
                buf0[i, j // 8, pl.ds((j % 8) * 16, 16)] = zer

        zbase = s * z_per_tile
        zleft = z_per_tile
        while zleft > 0:
            zn = min(zleft, chunk)
            pltpu.sync_copy(buf0.at[pl.ds(0, zn)],
                            acc.at[pl.ds(zbase + (z_per_tile - zleft), zn)])
            zleft -= zn
        plsc.subcore_barrier()

        trash = half + s * 8 + (lax.iota(jnp.int32, 16) % 8)

        @pl.loop(0, n_sup)
        def _(g):
            base = s * chunks_per_tile + g * sup
            pltpu.sync_copy(src_hbm.at[pl.ds(base, sup)], sidx)
            pltpu.sync_copy(dst_hbm.at[pl.ds(base, sup)], didx)

            @pl.loop(0, sup)
            def _(i):
                for j in range(chunk // 16):
                    d = didx[i, pl.ds(j * 16, 16)] - c * half
                    ok = (d >= 0) & (d < half)
                    didx[i, pl.ds(j * 16, 16)] = jnp.where(ok, d, trash)

            @pl.loop(0, sup)
            def _(k):
                pltpu.async_copy(xt_hbm.at[sidx.at[k]], buf0, sem0)
                pltpu.make_async_copy(xt_hbm.at[sidx.at[k]], buf0,
                                      sem0).wait()
                pltpu.sync_copy(buf0, acc.at[didx.at[k]], add=True)

        plsc.subcore_barrier()
        pltpu.sync_copy(acc.at[pl.ds(s * rows_per_tile, rows_per_tile)],
                        out_hbm.at[pl.ds(c * half + s * rows_per_tile,
                                         rows_per_tile)])

    return seg_kernel(xt.reshape(npad, 2, 128), src, dst)


def kernel(x, edge_index, W1, b1, Wd, bd):
    n, d_in = x.shape
    d_hid = W1.shape[0]
    d_out = Wd.shape[0]
    e = edge_index.shape[1]

    npad = ((n + 1279) // 1280) * 1280
    chunk = 128
    sup = 8
    per = 16 * chunk * sup
    chunks_per_tile = ((e + per - 1) // per) * sup
    e_pad = 16 * chunks_per_tile * chunk

    blk = 1280
    grid = npad // blk

    xpad = jnp.pad(x, ((0, npad - n), (0, 0)))
    w1b = W1[:, 1:]
    b1r = b1.reshape(1, d_hid)
    bdr = bd.reshape(1, d_out)

    xt = pl.pallas_call(
        _encoder_body,
        grid=(grid,),
        in_specs=[
            pl.BlockSpec((blk, d_in), lambda i: (i, 0)),
            pl.BlockSpec((d_hid, d_in), lambda i: (0, 0)),
            pl.BlockSpec((1, d_hid), lambda i: (0, 0)),
        ],
        out_specs=pl.BlockSpec((blk, d_hid), lambda i: (i, 0)),
        out_shape=jax.ShapeDtypeStruct((npad, d_hid), jnp.float32),
    )(xpad, w1b, b1r)

    src = jnp.concatenate([edge_index[1],
                           jnp.zeros((e_pad - e,), jnp.int32)])
    dst = jnp.concatenate([edge_index[0],
                           jnp.full((e_pad - e,), n, jnp.int32)])
    src = src.reshape(e_pad // chunk, chunk)
    dst = dst.reshape(e_pad // chunk, chunk)

    supp = _seg_sum_sc(xt, src, dst, npad, chunks_per_tile, chunk,
                       sup).reshape(npad, 256)

    out = pl.pallas_call(
        _decoder_body,
        grid=(grid,),
        in_specs=[
            pl.BlockSpec((blk, d_hid), lambda i: (i, 0)),
            pl.BlockSpec((d_out, d_hid), lambda i: (0, 0)),
            pl.BlockSpec((1, d_out), lambda i: (0, 0)),
        ],
        out_specs=pl.BlockSpec((blk, d_out), lambda i: (i, 0)),
        out_shape=jax.ShapeDtypeStruct((npad, d_out), jnp.float32),
    )(supp, Wd, bdr)

    return out[:n]


# speedup vs baseline: 1.4866x; 1.4866x over previous
import functools

import jax
import jax.numpy as jnp
from jax import lax
from jax.experimental import pallas as pl
from jax.experimental.pallas import tpu as pltpu
from jax.experimental.pallas import tpu_sc as plsc

MIN_NORM = 1e-15
EPS = 4e-3
MAX_NORM = 1e6


def _cosh(v):
    return 0.5 * (jnp.exp(v) + jnp.exp(-v))


def _sinh(v):
    return 0.5 * (jnp.exp(v) - jnp.exp(-v))


def _acosh(v):
    return jnp.log(v + jnp.sqrt(v * v - 1.0))


def _tailsq(v):
    return jnp.sum(v * v, axis=-1, keepdims=True) - v[:, 0:1] * v[:, 0:1]


def _proj_row(v, mask0):
    t = jnp.sqrt(jnp.clip(1.0 + _tailsq(v), EPS, None))
    return jnp.where(mask0, t, v)


def _expmap0_row(v, mask0):
    xn = jnp.sqrt(jnp.clip(_tailsq(v), MIN_NORM, None))
    w = (_sinh(xn) / xn) * v
    w = jnp.where(mask0, _cosh(xn), w)
    return _proj_row(w, mask0)


def _logmap0_row(v, mask0):
    yn = jnp.sqrt(jnp.clip(_tailsq(v), MIN_NORM, None))
    theta = jnp.clip(v[:, 0:1], 1.0 + EPS, None)
    w = (_acosh(theta) / yn) * v
    return jnp.where(mask0, 0.0, w)


def _encoder_body(x_ref, w_ref, b_ref, o_ref):
    x = x_ref[...]
    xn = jnp.sqrt(jnp.clip(jnp.sum(x * x, -1, keepdims=True), MIN_NORM, None))
    ys = (_sinh(xn) / xn) * x
    ysq = jnp.sum(ys * ys, -1, keepdims=True)
    t = jnp.sqrt(jnp.clip(1.0 + ysq, EPS, None))
    yn = jnp.sqrt(jnp.clip(ysq, MIN_NORM, None))
    theta = jnp.clip(t, 1.0 + EPS, None)
    ub = (_acosh(theta) / yn) * ys
    mm = lax.dot_general(ub, w_ref[...], (((1,), (1,)), ((), ())),
                         preferred_element_type=jnp.float32)
    mask0 = lax.broadcasted_iota(jnp.int32, mm.shape, 1) == 0
    res = _expmap0_row(mm, mask0)
    b = b_ref[...]
    bmask = lax.broadcasted_iota(jnp.int32, b.shape, 1) == 0
    hb = _expmap0_row(jnp.where(bmask, 0.0, b), bmask)
    lb = _logmap0_row(hb, bmask)
    x0 = res[:, 0:1]
    yn2 = jnp.sqrt(jnp.clip(_tailsq(res), MIN_NORM, None))
    alpha = jnp.sum(res * lb, -1, keepdims=True) / yn2
    vvec = jnp.where(mask0, -yn2, ((1.0 - x0) / yn2) * res)
    w = lb - alpha * vvec
    ux = jnp.sum(res * w, -1, keepdims=True) - x0 * w[:, 0:1]
    v0 = ux / jnp.clip(x0, EPS, None)
    u = jnp.where(mask0, v0, w)
    mdot = jnp.sum(u * u, -1, keepdims=True) - 2.0 * u[:, 0:1] * u[:, 0:1]
    normu = jnp.clip(jnp.sqrt(jnp.clip(mdot, EPS, None)), None, MAX_NORM)
    th = jnp.clip(normu, MIN_NORM, None)
    res2 = _proj_row(_cosh(th) * res + (_sinh(th) / th) * u, mask0)
    res2 = _proj_row(res2, mask0)
    xt = _logmap0_row(res2, mask0)
    o_ref[0] = xt[:, :128]
    o_ref[1] = xt[:, 128:]


def _decoder_body(s_ref, wd_ref, bd_ref, o_ref):
    sup = jnp.concatenate([s_ref[0], s_ref[1]], axis=-1)
    mask0 = lax.broadcasted_iota(jnp.int32, sup.shape, 1) == 0
    h = _expmap0_row(sup, mask0)
    lg = _logmap0_row(h, mask0)
    r = jnp.where(mask0, 0.0, jnp.maximum(lg, 0.0))
    h2 = _expmap0_row(r, mask0)
    hd = _logmap0_row(h2, mask0)
    out = lax.dot_general(hd, wd_ref[...], (((1,), (1,)), ((), ())),
                          preferred_element_type=jnp.float32) + bd_ref[...]
    m = jnp.max(out, -1, keepdims=True)
    z = out - m
    o_ref[...] = z - jnp.log(jnp.sum(jnp.exp(z), -1, keepdims=True))


def _seg_sum_sc(xt2, src, dst, npad, chunks_per_tile, chunk):
    rows_per_tile = npad // 16
    zcopies = rows_per_tile // chunk
    sup = 16
    n_sup = chunks_per_tile // sup
    edges_per_tile = chunks_per_tile * chunk
    mesh = plsc.VectorSubcoreMesh(core_axis_name="c", subcore_axis_name="s")

    @functools.partial(
        pl.kernel,
        out_type=jax.ShapeDtypeStruct((2 * npad, 128), jnp.float32),
        mesh=mesh,
        scratch_types=[
            pltpu.VMEM((sup, chunk), jnp.int32),
            pltpu.VMEM((sup, chunk), jnp.int32),
            pltpu.VMEM((chunk, 128), jnp.float32),
            pltpu.VMEM((chunk, 128), jnp.float32),
            pltpu.VMEM_SHARED((npad, 128), jnp.float32),
            pltpu.SemaphoreType.DMA,
            pltpu.SemaphoreType.DMA,
        ],
    )
    def seg_kernel(xt_hbm, src_hbm, dst_hbm, out_hbm, sidx, didx, buf0, buf1,
                   acc, sem0, sem1):
        c = lax.axis_index("c")
        s = lax.axis_index("s")
        zeros16 = jnp.zeros((16,), jnp.float32)

        @pl.loop(0, chunk)
        def _(i):
            for j in range(8):
                buf0[i, pl.ds(j * 16, 16)] = zeros16

        for m in range(zcopies):
            pltpu.sync_copy(buf0,
                            acc.at[pl.ds(s * rows_per_tile + m * chunk,
                                         chunk)])
        plsc.subcore_barrier()

        off = c * npad

        @pl.loop(0, n_sup)
        def _(g):
            base = s * chunks_per_tile + g * sup
            pltpu.sync_copy(src_hbm.at[pl.ds(base, sup)], sidx)
            pltpu.sync_copy(dst_hbm.at[pl.ds(base, sup)], didx)

            @pl.loop(0, sup)
            def _(i):
                for j in range(chunk // 16):
                    sidx[i, pl.ds(j * 16, 16)] = (
                        sidx[i, pl.ds(j * 16, 16)] + off)

            @pl.loop(0, sup)
            def _(k):
                pltpu.async_copy(xt_hbm.at[sidx.at[k]], buf0, sem0)
                pltpu.make_async_copy(xt_hbm.at[sidx.at[k]], buf0,
                                      sem0).wait()
                pltpu.sync_copy(buf0, acc.at[didx.at[k]], add=True)

        plsc.subcore_barrier()
        pltpu.sync_copy(acc.at[pl.ds(s * rows_per_tile, rows_per_tile)],
                        out_hbm.at[pl.ds(off + s * rows_per_tile,
                                         rows_per_tile)])

    return seg_kernel(xt2, src, dst)


def kernel(x, edge_index, W1, b1, Wd, bd):
    n, d_in = x.shape
    d_hid = W1.shape[0]
    d_out = Wd.shape[0]
    e = edge_index.shape[1]

    npad = ((n + 1279) // 1280) * 1280
    chunk = 128
    chunks_per_tile = (e + 16 * chunk - 1) // (16 * chunk)
    chunks_per_tile = ((chunks_per_tile + 15) // 16) * 16
    e_pad = 16 * chunks_per_tile * chunk

    blk = 1280
    grid = npad // blk

    xpad = jnp.pad(x, ((0, npad - n), (0, 0)))
    w1b = W1[:, 1:]
    b1r = b1.reshape(1, d_hid)
    bdr = bd.reshape(1, d_out)

    xt2 = pl.pallas_call(
        _encoder_body,
        grid=(grid,),
        in_specs=[
            pl.BlockSpec((blk, d_in), lambda i: (i, 0)),
            pl.BlockSpec((d_hid, d_in), lambda i: (0, 0)),
            pl.BlockSpec((1, d_hid), lambda i: (0, 0)),
        ],
        out_specs=pl.BlockSpec((2, blk, 128), lambda i: (0, i, 0)),
        out_shape=jax.ShapeDtypeStruct((2, npad, 128), jnp.float32),
    )(xpad, w1b, b1r)

    src = jnp.concatenate([edge_index[1],
                           jnp.zeros((e_pad - e,), jnp.int32)])
    dst = jnp.concatenate([edge_index[0],
                           jnp.full((e_pad - e,), n, jnp.int32)])
    src = src.reshape(e_pad // chunk, chunk)
    dst = dst.reshape(e_pad // chunk, chunk)

    supp = _seg_sum_sc(xt2.reshape(2 * npad, 128), src, dst, npad,
                       chunks_per_tile, chunk)

    out = pl.pallas_call(
        _decoder_body,
        grid=(grid,),
        in_specs=[
            pl.BlockSpec((2, blk, 128), lambda i: (0, i, 0)),
            pl.BlockSpec((d_out, d_hid), lambda i: (0, 0)),
            pl.BlockSpec((1, d_out), lambda i: (0, 0)),
        ],
        out_specs=pl.BlockSpec((blk, d_out), lambda i: (i, 0)),
        out_shape=jax.ShapeDtypeStruct((npad, d_out), jnp.float32),
    )(supp.reshape(2, npad, 128), Wd, bdr)

    return out[:n]


# speedup vs baseline: 1.8637x; 1.2536x over previous
import functools

import jax
import jax.numpy as jnp
from jax import lax
from jax.experimental import pallas as pl
from jax.experimental.pallas import tpu as pltpu
from jax.experimental.pallas import tpu_sc as plsc

MIN_NORM = 1e-15
EPS = 4e-3
MAX_NORM = 1e6


def _cosh(v):
    return 0.5 * (jnp.exp(v) + jnp.exp(-v))


def _sinh(v):
    return 0.5 * (jnp.exp(v) - jnp.exp(-v))


def _acosh(v):
    return jnp.log(v + jnp.sqrt(v * v - 1.0))


def _tailsq(v):
    return jnp.sum(v * v, axis=-1, keepdims=True) - v[:, 0:1] * v[:, 0:1]


def _proj_row(v, mask0):
    t = jnp.sqrt(jnp.clip(1.0 + _tailsq(v), EPS, None))
    return jnp.where(mask0, t, v)


def _expmap0_row(v, mask0):
    xn = jnp.sqrt(jnp.clip(_tailsq(v), MIN_NORM, None))
    w = (_sinh(xn) / xn) * v
    w = jnp.where(mask0, _cosh(xn), w)
    return _proj_row(w, mask0)


def _logmap0_row(v, mask0):
    yn = jnp.sqrt(jnp.clip(_tailsq(v), MIN_NORM, None))
    theta = jnp.clip(v[:, 0:1], 1.0 + EPS, None)
    w = (_acosh(theta) / yn) * v
    return jnp.where(mask0, 0.0, w)


def _encoder_body(x_ref, w_ref, b_ref, o_ref):
    x = x_ref[...]
    xn = jnp.sqrt(jnp.clip(jnp.sum(x * x, -1, keepdims=True), MIN_NORM, None))
    ys = (_sinh(xn) / xn) * x
    ysq = jnp.sum(ys * ys, -1, keepdims=True)
    t = jnp.sqrt(jnp.clip(1.0 + ysq, EPS, None))
    yn = jnp.sqrt(jnp.clip(ysq, MIN_NORM, None))
    theta = jnp.clip(t, 1.0 + EPS, None)
    ub = (_acosh(theta) / yn) * ys
    mm = lax.dot_general(ub, w_ref[...], (((1,), (1,)), ((), ())),
                         preferred_element_type=jnp.float32)
    mask0 = lax.broadcasted_iota(jnp.int32, mm.shape, 1) == 0
    res = _expmap0_row(mm, mask0)
    b = b_ref[...]
    bmask = lax.broadcasted_iota(jnp.int32, b.shape, 1) == 0
    hb = _expmap0_row(jnp.where(bmask, 0.0, b), bmask)
    lb = _logmap0_row(hb, bmask)
    x0 = res[:, 0:1]
    yn2 = jnp.sqrt(jnp.clip(_tailsq(res), MIN_NORM, None))
    alpha = jnp.sum(res * lb, -1, keepdims=True) / yn2
    vvec = jnp.where(mask0, -yn2, ((1.0 - x0) / yn2) * res)
    w = lb - alpha * vvec
    ux = jnp.sum(res * w, -1, keepdims=True) - x0 * w[:, 0:1]
    v0 = ux / jnp.clip(x0, EPS, None)
    u = jnp.where(mask0, v0, w)
    mdot = jnp.sum(u * u, -1, keepdims=True) - 2.0 * u[:, 0:1] * u[:, 0:1]
    normu = jnp.clip(jnp.sqrt(jnp.clip(mdot, EPS, None)), None, MAX_NORM)
    th = jnp.clip(normu, MIN_NORM, None)
    res2 = _proj_row(_cosh(th) * res + (_sinh(th) / th) * u, mask0)
    res2 = _proj_row(res2, mask0)
    xt = _logmap0_row(res2, mask0)
    o_ref[0] = xt[:, :128]
    o_ref[1] = xt[:, 128:]


def _decoder_body(s_ref, wd_ref, bd_ref, o_ref):
    sup = jnp.concatenate([s_ref[0], s_ref[1]], axis=-1)
    mask0 = lax.broadcasted_iota(jnp.int32, sup.shape, 1) == 0
    h = _expmap0_row(sup, mask0)
    lg = _logmap0_row(h, mask0)
    r = jnp.where(mask0, 0.0, jnp.maximum(lg, 0.0))
    h2 = _expmap0_row(r, mask0)
    hd = _logmap0_row(h2, mask0)
    out = lax.dot_general(hd, wd_ref[...], (((1,), (1,)), ((), ())),
                          preferred_element_type=jnp.float32) + bd_ref[...]
    m = jnp.max(out, -1, keepdims=True)
    z = out - m
    o_ref[...] = z - jnp.log(jnp.sum(jnp.exp(z), -1, keepdims=True))


def _seg_sum_sc(xt2, src, dst, npad, chunks_per_tile, chunk):
    rows_per_tile = npad // 16
    zcopies = rows_per_tile // chunk
    edges_per_tile = chunks_per_tile * chunk
    mesh = plsc.VectorSubcoreMesh(core_axis_name="c", subcore_axis_name="s")

    @functools.partial(
        pl.kernel,
        out_type=jax.ShapeDtypeStruct((2 * npad, 128), jnp.float32),
        mesh=mesh,
        scratch_types=[
            pltpu.VMEM((chunk,), jnp.int32),
            pltpu.VMEM((chunk,), jnp.int32),
            pltpu.VMEM((chunk, 128), jnp.float32),
            pltpu.VMEM_SHARED((npad, 128), jnp.float32),
            pltpu.SemaphoreType.DMA,
        ],
    )
    def seg_kernel(xt_hbm, src_hbm, dst_hbm, out_hbm, sidx, didx, rows,
                   acc, sem):
        c = lax.axis_index("c")
        s = lax.axis_index("s")
        zeros16 = jnp.zeros((16,), jnp.float32)

        @pl.loop(0, chunk)
        def _(i):
            for j in range(8):
                rows[i, pl.ds(j * 16, 16)] = zeros16

        for m in range(zcopies):
            pltpu.sync_copy(rows,
                            acc.at[pl.ds(s * rows_per_tile + m * chunk,
                                         chunk)])
        plsc.subcore_barrier()

        off = c * npad

        @pl.loop(0, chunks_per_tile)
        def _(k):
            base = s * edges_per_tile + k * chunk
            pltpu.sync_copy(src_hbm.at[pl.ds(base, chunk)], sidx)
            pltpu.sync_copy(dst_hbm.at[pl.ds(base, chunk)], didx)
            for j in range(chunk // 16):
                sidx[pl.ds(j * 16, 16)] = sidx[pl.ds(j * 16, 16)] + off
            pltpu.async_copy(xt_hbm.at[sidx], rows, sem).wait()
            pltpu.sync_copy(rows, acc.at[didx], add=True)

        plsc.subcore_barrier()
        pltpu.sync_copy(acc.at[pl.ds(s * rows_per_tile, rows_per_tile)],
                        out_hbm.at[pl.ds(off + s * rows_per_tile,
                                         rows_per_tile)])

    return seg_kernel(xt2, src, dst)


def kernel(x, edge_index, W1, b1, Wd, bd):
    n, d_in = x.shape
    d_hid = W1.shape[0]
    d_out = Wd.shape[0]
    e = edge_index.shape[1]

    npad = ((n + 1279) // 1280) * 1280
    chunk = 128
    chunks_per_tile = (e + 16 * chunk - 1) // (16 * chunk)
    e_pad = 16 * chunks_per_tile * chunk

    blk = 1280
    grid = npad // blk

    xpad = jnp.pad(x, ((0, npad - n), (0, 0)))
    w1b = W1[:, 1:]
    b1r = b1.reshape(1, d_hid)
    bdr = bd.reshape(1, d_out)

    xt2 = pl.pallas_call(
        _encoder_body,
        grid=(grid,),
        in_specs=[
            pl.BlockSpec((blk, d_in), lambda i: (i, 0)),
            pl.BlockSpec((d_hid, d_in), lambda i: (0, 0)),
            pl.BlockSpec((1, d_hid), lambda i: (0, 0)),
        ],
        out_specs=pl.BlockSpec((2, blk, 128), lambda i: (0, i, 0)),
        out_shape=jax.ShapeDtypeStruct((2, npad, 128), jnp.float32),
    )(xpad, w1b, b1r)

    src = jnp.concatenate([edge_index[1],
                           jnp.zeros((e_pad - e,), jnp.int32)])
    dst = jnp.concatenate([edge_index[0],
                           jnp.full((e_pad - e,), n, jnp.int32)])
    supp = _seg_sum_sc(xt2.reshape(2 * npad, 128), src, dst, npad,
                       chunks_per_tile, chunk)

    out = pl.pallas_call(
        _decoder_body,
        grid=(grid,),
        in_specs=[
            pl.BlockSpec((2, blk, 128), lambda i: (0, i, 0)),
            pl.BlockSpec((d_out, d_hid), lambda i: (0, 0)),
            pl.BlockSpec((1, d_out), lambda i: (0, 0)),
        ],
        out_specs=pl.BlockSpec((blk, d_out), lambda i: (i, 0)),
        out_shape=jax.ShapeDtypeStruct((npad, d_out), jnp.float32),
    )(supp.reshape(2, npad, 128), Wd, bdr)

    return out[:n]


# speedup vs baseline: 2.1962x; 1.1784x over previous
import functools

import jax
import jax.numpy as jnp
from jax import lax
from jax.experimental import pallas as pl
from jax.experimental.pallas import tpu as pltpu
from jax.experimental.pallas import tpu_sc as plsc

MIN_NORM = 1e-15
EPS = 4e-3
MAX_NORM = 1e6


def _cosh(v):
    return 0.5 * (jnp.exp(v) + jnp.exp(-v))


def _sinh(v):
    return 0.5 * (jnp.exp(v) - jnp.exp(-v))


def _acosh(v):
    return jnp.log(v + jnp.sqrt(v * v - 1.0))


def _tailsq(v):
    return jnp.sum(v * v, axis=-1, keepdims=True) - v[:, 0:1] * v[:, 0:1]


def _proj_row(v, mask0):
    t = jnp.sqrt(jnp.clip(1.0 + _tailsq(v), EPS, None))
    return jnp.where(mask0, t, v)


def _expmap0_row(v, mask0):
    xn = jnp.sqrt(jnp.clip(_tailsq(v), MIN_NORM, None))
    w = (_sinh(xn) / xn) * v
    w = jnp.where(mask0, _cosh(xn), w)
    return _proj_row(w, mask0)


def _logmap0_row(v, mask0):
    yn = jnp.sqrt(jnp.clip(_tailsq(v), MIN_NORM, None))
    theta = jnp.clip(v[:, 0:1], 1.0 + EPS, None)
    w = (_acosh(theta) / yn) * v
    return jnp.where(mask0, 0.0, w)


def _encoder_body(x_ref, w_ref, b_ref, o_ref):
    x = x_ref[...]
    xn = jnp.sqrt(jnp.clip(jnp.sum(x * x, -1, keepdims=True), MIN_NORM, None))
    ys = (_sinh(xn) / xn) * x
    ysq = jnp.sum(ys * ys, -1, keepdims=True)
    t = jnp.sqrt(jnp.clip(1.0 + ysq, EPS, None))
    yn = jnp.sqrt(jnp.clip(ysq, MIN_NORM, None))
    theta = jnp.clip(t, 1.0 + EPS, None)
    ub = (_acosh(theta) / yn) * ys
    mm = lax.dot_general(ub, w_ref[...], (((1,), (1,)), ((), ())),
                         preferred_element_type=jnp.float32)
    mask0 = lax.broadcasted_iota(jnp.int32, mm.shape, 1) == 0
    res = _expmap0_row(mm, mask0)
    b = b_ref[...]
    bmask = lax.broadcasted_iota(jnp.int32, b.shape, 1) == 0
    hb = _expmap0_row(jnp.where(bmask, 0.0, b), bmask)
    lb = _logmap0_row(hb, bmask)
    x0 = res[:, 0:1]
    yn2 = jnp.sqrt(jnp.clip(_tailsq(res), MIN_NORM, None))
    alpha = jnp.sum(res * lb, -1, keepdims=True) / yn2
    vvec = jnp.where(mask0, -yn2, ((1.0 - x0) / yn2) * res)
    w = lb - alpha * vvec
    ux = jnp.sum(res * w, -1, keepdims=True) - x0 * w[:, 0:1]
    v0 = ux / jnp.clip(x0, EPS, None)
    u = jnp.where(mask0, v0, w)
    mdot = jnp.sum(u * u, -1, keepdims=True) - 2.0 * u[:, 0:1] * u[:, 0:1]
    normu = jnp.clip(jnp.sqrt(jnp.clip(mdot, EPS, None)), None, MAX_NORM)
    th = jnp.clip(normu, MIN_NORM, None)
    res2 = _proj_row(_cosh(th) * res + (_sinh(th) / th) * u, mask0)
    res2 = _proj_row(res2, mask0)
    xt = _logmap0_row(res2, mask0)
    o_ref[0] = xt[:, :128]
    o_ref[1] = xt[:, 128:]


def _decoder_body(s_ref, wd_ref, bd_ref, o_ref):
    sup = jnp.concatenate([s_ref[0], s_ref[1]], axis=-1)
    mask0 = lax.broadcasted_iota(jnp.int32, sup.shape, 1) == 0
    h = _expmap0_row(sup, mask0)
    lg = _logmap0_row(h, mask0)
    r = jnp.where(mask0, 0.0, jnp.maximum(lg, 0.0))
    h2 = _expmap0_row(r, mask0)
    hd = _logmap0_row(h2, mask0)
    out = lax.dot_general(hd, wd_ref[...], (((1,), (1,)), ((), ())),
                          preferred_element_type=jnp.float32) + bd_ref[...]
    m = jnp.max(out, -1, keepdims=True)
    z = out - m
    o_ref[...] = z - jnp.log(jnp.sum(jnp.exp(z), -1, keepdims=True))


def _seg_sum_sc(xt2, src, dst, npad, chunks_per_tile, chunk):
    rows_per_tile = npad // 16
    zcopies = rows_per_tile // chunk
    edges_per_tile = chunks_per_tile * chunk
    mesh = plsc.VectorSubcoreMesh(core_axis_name="c", subcore_axis_name="s")

    @functools.partial(
        pl.kernel,
        out_type=jax.ShapeDtypeStruct((2 * npad, 128), jnp.float32),
        mesh=mesh,
        scratch_types=[
            pltpu.VMEM((chunk,), jnp.int32),
            pltpu.VMEM((chunk,), jnp.int32),
            pltpu.VMEM((chunk,), jnp.int32),
            pltpu.VMEM((chunk,), jnp.int32),
            pltpu.VMEM((chunk, 128), jnp.float32),
            pltpu.VMEM((chunk, 128), jnp.float32),
            pltpu.VMEM_SHARED((npad, 128), jnp.float32),
            pltpu.SemaphoreType.DMA,
            pltpu.SemaphoreType.DMA,
        ],
    )
    def seg_kernel(xt_hbm, src_hbm, dst_hbm, out_hbm, sidxa, didxa, sidxb,
                   didxb, rows, rowsb, acc, sema, semb):
        c = lax.axis_index("c")
        s = lax.axis_index("s")
        zeros16 = jnp.zeros((16,), jnp.float32)

        @pl.loop(0, chunk)
        def _(i):
            for j in range(8):
                rows[i, pl.ds(j * 16, 16)] = zeros16

        for m in range(zcopies):
            pltpu.sync_copy(rows,
                            acc.at[pl.ds(s * rows_per_tile + m * chunk,
                                         chunk)])
        plsc.subcore_barrier()

        off = c * npad
        tbase = s * edges_per_tile

        def load(k, si, di):
            pltpu.sync_copy(src_hbm.at[pl.ds(tbase + k * chunk, chunk)], si)
            pltpu.sync_copy(dst_hbm.at[pl.ds(tbase + k * chunk, chunk)], di)
            for j in range(chunk // 16):
                si[pl.ds(j * 16, 16)] = si[pl.ds(j * 16, 16)] + off

        load(0, sidxa, didxa)
        pltpu.async_copy(xt_hbm.at[sidxa], rows, sema)

        @pl.loop(0, chunks_per_tile, step=2)
        def _(k):
            load(k + 1, sidxb, didxb)
            pltpu.async_copy(xt_hbm.at[sidxb], rowsb, semb)
            pltpu.make_async_copy(xt_hbm.at[sidxa], rows, sema).wait()
            pltpu.sync_copy(rows, acc.at[didxa], add=True)

            @pl.when(k + 2 < chunks_per_tile)
            def _():
                load(k + 2, sidxa, didxa)
                pltpu.async_copy(xt_hbm.at[sidxa], rows, sema)

            pltpu.make_async_copy(xt_hbm.at[sidxb], rowsb, semb).wait()
            pltpu.sync_copy(rowsb, acc.at[didxb], add=True)

        plsc.subcore_barrier()
        pltpu.sync_copy(acc.at[pl.ds(s * rows_per_tile, rows_per_tile)],
                        out_hbm.at[pl.ds(off + s * rows_per_tile,
                                         rows_per_tile)])

    return seg_kernel(xt2, src, dst)


def kernel(x, edge_index, W1, b1, Wd, bd):
    n, d_in = x.shape
    d_hid = W1.shape[0]
    d_out = Wd.shape[0]
    e = edge_index.shape[1]

    npad = ((n + 1279) // 1280) * 1280
    chunk = 128
    chunks_per_tile = (e + 16 * chunk - 1) // (16 * chunk)
    chunks_per_tile += chunks_per_tile % 2
    e_pad = 16 * chunks_per_tile * chunk

    blk = 1280
    grid = npad // blk

    xpad = jnp.pad(x, ((0, npad - n), (0, 0)))
    w1b = W1[:, 1:]
    b1r = b1.reshape(1, d_hid)
    bdr = bd.reshape(1, d_out)

    xt2 = pl.pallas_call(
        _encoder_body,
        grid=(grid,),
        in_specs=[
            pl.BlockSpec((blk, d_in), lambda i: (i, 0)),
            pl.BlockSpec((d_hid, d_in), lambda i: (0, 0)),
            pl.BlockSpec((1, d_hid), lambda i: (0, 0)),
        ],
        out_specs=pl.BlockSpec((2, blk, 128), lambda i: (0, i, 0)),
        out_shape=jax.ShapeDtypeStruct((2, npad, 128), jnp.float32),
    )(xpad, w1b, b1r)

    src = jnp.concatenate([edge_index[1],
                           jnp.zeros((e_pad - e,), jnp.int32)])
    dst = jnp.concatenate([edge_index[0],
                           jnp.full((e_pad - e,), n, jnp.int32)])
    supp = _seg_sum_sc(xt2.reshape(2 * npad, 128), src, dst, npad,
                       chunks_per_tile, chunk)

    out = pl.pallas_call(
        _decoder_body,
        grid=(grid,),
        in_specs=[
            pl.BlockSpec((2, blk, 128), lambda i: (0, i, 0)),
            pl.BlockSpec((d_out, d_hid), lambda i: (0, 0)),
            pl.BlockSpec((1, d_out), lambda i: (0, 0)),
        ],
        out_specs=pl.BlockSpec((blk, d_out), lambda i: (i, 0)),
        out_shape=jax.ShapeDtypeStruct((npad, d_out), jnp.float32),
    )(supp.reshape(2, npad, 128), Wd, bdr)

    return out[:n]
